# Initial kernel scaffold; baseline (speedup 1.0000x reference)
import jax
import jax.numpy as jnp
from jax.experimental import pallas as pl


def kernel(x, edge_index, W1, b1, W2, b2):
    raise NotImplementedError("write your pallas kernel here")


# speedup vs baseline: 21.2357x; 21.2357x over previous
import functools

import jax
import jax.numpy as jnp
from jax import lax
from jax.experimental import pallas as pl
from jax.experimental.pallas import tpu as pltpu
from jax.experimental.pallas import tpu_sc as plsc

_NC = 2
_NS = 16
_CHUNK = 128
_W = 16


def _sc_degree(n_pad, n_edges_pad):
    workers = _NC * _NS
    epw = n_edges_pad // workers
    n_chunks = epw // _CHUNK
    mesh = plsc.VectorSubcoreMesh(core_axis_name="c", subcore_axis_name="s")

    @functools.partial(
        pl.kernel,
        out_type=jax.ShapeDtypeStruct((_NC, n_pad, _W), jnp.float32),
        mesh=mesh,
        compiler_params=pltpu.CompilerParams(use_tc_tiling_on_sc=False),
        scratch_types=[
            pltpu.VMEM((_CHUNK,), jnp.int32),
            pltpu.VMEM((_CHUNK, _W), jnp.float32),
            pltpu.VMEM_SHARED((n_pad, _W), jnp.float32),
        ],
    )
    def deg_kernel(ones_hbm, dst_hbm, init_hbm, out_hbm, didx, rows, acc):
        c = lax.axis_index("c")
        s = lax.axis_index("s")

        @pl.when(s == 0)
        def _():
            pltpu.sync_copy(init_hbm, acc)

        pltpu.sync_copy(ones_hbm, rows)
        plsc.subcore_barrier()

        base = (c * _NS + s) * epw

        def step(i, carry):
            off = pl.multiple_of(base + i * _CHUNK, _CHUNK)
            pltpu.sync_copy(dst_hbm.at[pl.ds(off, _CHUNK)], didx)
            pltpu.sync_copy(rows, acc.at[didx], add=True)
            return carry

        lax.fori_loop(0, n_chunks, step, 0)
        plsc.subcore_barrier()

        @pl.when(s == 0)
        def _():
            pltpu.sync_copy(acc, out_hbm.at[c])

    return deg_kernel


def _sc_scatter(n_pad, n_edges_pad):
    workers = _NC * _NS
    epw = n_edges_pad // workers
    n_chunks = epw // _CHUNK
    mesh = plsc.VectorSubcoreMesh(core_axis_name="c", subcore_axis_name="s")

    @functools.partial(
        pl.kernel,
        out_type=jax.ShapeDtypeStruct((_NC, n_pad, _W), jnp.float32),
        mesh=mesh,
        compiler_params=pltpu.CompilerParams(use_tc_tiling_on_sc=False),
        scratch_types=[
            pltpu.VMEM((_CHUNK,), jnp.int32),
            pltpu.VMEM((_CHUNK,), jnp.int32),
            pltpu.VMEM((_CHUNK, _W), jnp.float32),
            pltpu.VMEM_SHARED((n_pad, _W), jnp.float32),
            pltpu.SemaphoreType.DMA,
        ],
    )
    def scat_kernel(table_hbm, src_hbm, dst_hbm, init_hbm, out_hbm,
                    sidx, didx, rows, acc, sem):
        c = lax.axis_index("c")
        s = lax.axis_index("s")

        @pl.when(s == 0)
        def _():
            pltpu.sync_copy(init_hbm, acc)

        plsc.subcore_barrier()

        base = (c * _NS + s) * epw

        def step(i, carry):
            off = pl.multiple_of(base + i * _CHUNK, _CHUNK)
            pltpu.sync_copy(src_hbm.at[pl.ds(off, _CHUNK)], sidx)
            pltpu.async_copy(table_hbm.at[sidx], rows, sem).wait()
            pltpu.sync_copy(dst_hbm.at[pl.ds(off, _CHUNK)], didx)
            pltpu.sync_copy(rows, acc.at[didx], add=True)
            return carry

        lax.fori_loop(0, n_chunks, step, 0)
        plsc.subcore_barrier()

        @pl.when(s == 0)
        def _():
            pltpu.sync_copy(acc, out_hbm.at[c])

    return scat_kernel


def _tc_prep_body(degp_ref, x_ref, w1_ref, p1_ref, dinv_ref):
    n = x_ref.shape[0]
    deg = degp_ref[0, :n, 0:1] + degp_ref[1, :n, 0:1] + 1.0
    dinv = lax.rsqrt(deg)
    h1 = jnp.dot(x_ref[...], w1_ref[...], preferred_element_type=jnp.float32)
    p1_ref[...] = jnp.zeros(p1_ref.shape, jnp.float32)
    p1_ref[:n, 0:2] = h1 * dinv
    dinv_ref[...] = dinv


def _tc_mid_body(s1_ref, p1_ref, dinv_ref, b1_ref, p2_ref):
    n = dinv_ref.shape[0]
    stot = s1_ref[0, :n, 0:2] + s1_ref[1, :n, 0:2] + p1_ref[:n, 0:2]
    z = jnp.maximum(dinv_ref[...] * stot + b1_ref[...], 0.0)
    p2_ref[...] = jnp.zeros(p2_ref.shape, jnp.float32)
    p2_ref[:n, 0:2] = dinv_ref[...] * z


def _tc_final_body(s2_ref, p2_ref, dinv_ref, w2_ref, b2_ref, out_ref):
    n = dinv_ref.shape[0]
    agg = dinv_ref[...] * (
        s2_ref[0, :n, 0:2] + s2_ref[1, :n, 0:2] + p2_ref[:n, 0:2])
    y = jnp.dot(agg, w2_ref[...], preferred_element_type=jnp.float32)
    y = y + b2_ref[...]
    m = jnp.max(y, axis=-1, keepdims=True)
    e = jnp.exp(y - m)
    out_ref[...] = (y - m) - jnp.log(jnp.sum(e, axis=-1, keepdims=True))


def kernel(x, edge_index, W1, b1, W2, b2):
    n, d_in = x.shape
    e = edge_index.shape[1]
    n_pad = n + 8
    step = _NC * _NS * _CHUNK
    e_pad = ((e + step - 1) // step) * step

    src = edge_index[0]
    dst = edge_index[1]
    fill = (jnp.arange(e_pad - e, dtype=src.dtype) % 8) + n
    src_p = jnp.concatenate([src, fill])
    dst_p = jnp.concatenate([dst, fill])

    zeros_nw = jnp.zeros((n_pad, _W), jnp.float32)
    ones = jnp.ones((_CHUNK, _W), jnp.float32)

    deg_parts = _sc_degree(n_pad, e_pad)(ones, dst_p, zeros_nw)

    p1p, dinv = pl.pallas_call(
        _tc_prep_body,
        out_shape=(jax.ShapeDtypeStruct((n_pad, _W), jnp.float32),
                   jax.ShapeDtypeStruct((n, 1), jnp.float32)),
    )(deg_parts, x, W1)

    scat = _sc_scatter(n_pad, e_pad)
    s1 = scat(p1p, src_p, dst_p, zeros_nw)

    p2p = pl.pallas_call(
        _tc_mid_body,
        out_shape=jax.ShapeDtypeStruct((n_pad, _W), jnp.float32),
    )(s1, p1p, dinv, b1.reshape(1, 2))

    s2 = scat(p2p, src_p, dst_p, zeros_nw)

    out = pl.pallas_call(
        _tc_final_body,
        out_shape=jax.ShapeDtypeStruct((n, W2.shape[1]), jnp.float32),
    )(s2, p2p, dinv, W2, b2.reshape(1, W2.shape[1]))
    return out


# speedup vs baseline: 33.2193x; 1.5643x over previous
import functools

import jax
import jax.numpy as jnp
from jax import lax
from jax.experimental import pallas as pl
from jax.experimental.pallas import tpu as pltpu
from jax.experimental.pallas import tpu_sc as plsc

_NC = 2
_NS = 16
_CHUNK = 128
_W = 16


def _sc_degree(n_pad, n_chunks):
    mesh = plsc.VectorSubcoreMesh(core_axis_name="c", subcore_axis_name="s")

    @functools.partial(
        pl.kernel,
        out_type=jax.ShapeDtypeStruct((_NC, n_pad, _W), jnp.float32),
        mesh=mesh,
        compiler_params=pltpu.CompilerParams(use_tc_tiling_on_sc=False),
        scratch_types=[
            pltpu.VMEM((n_chunks, _CHUNK), jnp.int32),
            pltpu.VMEM((_CHUNK, _W), jnp.float32),
            pltpu.VMEM_SHARED((n_pad, _W), jnp.float32),
        ],
    )
    def deg_kernel(ones_hbm, dst_hbm, init_hbm, out_hbm, didx, rows, acc):
        c = lax.axis_index("c")
        s = lax.axis_index("s")

        @pl.when(s == 0)
        def _():
            pltpu.sync_copy(init_hbm, acc)

        w = c * _NS + s
        pltpu.sync_copy(dst_hbm.at[w], didx)
        pltpu.sync_copy(ones_hbm, rows)
        plsc.subcore_barrier()

        def step(i, carry):
            pltpu.sync_copy(rows, acc.at[didx.at[i]], add=True)
            return carry

        lax.fori_loop(0, n_chunks, step, 0)
        plsc.subcore_barrier()

        @pl.when(s == 0)
        def _():
            pltpu.sync_copy(acc, out_hbm.at[c])

    return deg_kernel


def _sc_scatter(n_pad, n_chunks):
    mesh = plsc.VectorSubcoreMesh(core_axis_name="c", subcore_axis_name="s")
    depth = 4
    assert n_chunks % depth == 0

    @functools.partial(
        pl.kernel,
        out_type=jax.ShapeDtypeStruct((_NC, n_pad, _W), jnp.float32),
        mesh=mesh,
        compiler_params=pltpu.CompilerParams(use_tc_tiling_on_sc=False),
        scratch_types=[
            pltpu.VMEM((n_chunks, _CHUNK), jnp.int32),
            pltpu.VMEM((n_chunks, _CHUNK), jnp.int32),
            [pltpu.VMEM((_CHUNK, _W), jnp.float32) for _ in range(depth)],
            [pltpu.SemaphoreType.DMA for _ in range(depth)],
            pltpu.VMEM_SHARED((n_pad, _W), jnp.float32),
        ],
    )
    def scat_kernel(table_hbm, src_hbm, dst_hbm, init_hbm, out_hbm,
                    sidx, didx, rows, sems, acc):
        c = lax.axis_index("c")
        s = lax.axis_index("s")

        @pl.when(s == 0)
        def _():
            pltpu.sync_copy(init_hbm, acc)

        w = c * _NS + s
        pltpu.sync_copy(src_hbm.at[w], sidx)
        pltpu.sync_copy(dst_hbm.at[w], didx)
        plsc.subcore_barrier()

        def step(q, carry):
            base = q * depth
            handles = []
            for k in range(depth):
                handles.append(pltpu.async_copy(
                    table_hbm.at[sidx.at[base + k]], rows[k], sems[k]))
            for k in range(depth):
                handles[k].wait()
                pltpu.sync_copy(rows[k], acc.at[didx.at[base + k]], add=True)
            return carry

        lax.fori_loop(0, n_chunks // depth, step, 0)
        plsc.subcore_barrier()

        @pl.when(s == 0)
        def _():
            pltpu.sync_copy(acc, out_hbm.at[c])

    return scat_kernel


def _tc_prep_body(degp_ref, x_ref, w1_ref, p1_ref, dinv_ref):
    n = x_ref.shape[0]
    deg = degp_ref[0, :n, 0:1] + degp_ref[1, :n, 0:1] + 1.0
    dinv = lax.rsqrt(deg)
    h1 = jnp.dot(x_ref[...], w1_ref[...], preferred_element_type=jnp.float32)
    p1_ref[...] = jnp.zeros(p1_ref.shape, jnp.float32)
    p1_ref[:n, 0:2] = h1 * dinv
    dinv_ref[...] = dinv


def _tc_mid_body(s1_ref, p1_ref, dinv_ref, b1_ref, p2_ref):
    n = dinv_ref.shape[0]
    stot = s1_ref[0, :n, 0:2] + s1_ref[1, :n, 0:2] + p1_ref[:n, 0:2]
    z = jnp.maximum(dinv_ref[...] * stot + b1_ref[...], 0.0)
    p2_ref[...] = jnp.zeros(p2_ref.shape, jnp.float32)
    p2_ref[:n, 0:2] = dinv_ref[...] * z


def _tc_final_body(s2_ref, p2_ref, dinv_ref, w2_ref, b2_ref, out_ref):
    n = dinv_ref.shape[0]
    agg = dinv_ref[...] * (
        s2_ref[0, :n, 0:2] + s2_ref[1, :n, 0:2] + p2_ref[:n, 0:2])
    y = jnp.dot(agg, w2_ref[...], preferred_element_type=jnp.float32)
    y = y + b2_ref[...]
    m = jnp.max(y, axis=-1, keepdims=True)
    e = jnp.exp(y - m)
    out_ref[...] = (y - m) - jnp.log(jnp.sum(e, axis=-1, keepdims=True))


def kernel(x, edge_index, W1, b1, W2, b2):
    n, d_in = x.shape
    e = edge_index.shape[1]
    n_pad = n + 8
    workers = _NC * _NS
    step = workers * _CHUNK * 4
    e_pad = ((e + step - 1) // step) * step
    n_chunks = e_pad // (workers * _CHUNK)

    src = edge_index[0]
    dst = edge_index[1]
    fill = (jnp.arange(e_pad - e, dtype=src.dtype) % 8) + n
    src_p = jnp.concatenate([src, fill]).reshape(workers, n_chunks, _CHUNK)
    dst_p = jnp.concatenate([dst, fill]).reshape(workers, n_chunks, _CHUNK)

    zeros_nw = jnp.zeros((n_pad, _W), jnp.float32)
    ones = jnp.ones((_CHUNK, _W), jnp.float32)

    deg_parts = _sc_degree(n_pad, n_chunks)(ones, dst_p, zeros_nw)

    p1p, dinv = pl.pallas_call(
        _tc_prep_body,
        out_shape=(jax.ShapeDtypeStruct((n_pad, _W), jnp.float32),
                   jax.ShapeDtypeStruct((n, 1), jnp.float32)),
    )(deg_parts, x, W1)

    scat = _sc_scatter(n_pad, n_chunks)
    s1 = scat(p1p, src_p, dst_p, zeros_nw)

    p2p = pl.pallas_call(
        _tc_mid_body,
        out_shape=jax.ShapeDtypeStruct((n_pad, _W), jnp.float32),
    )(s1, p1p, dinv, b1.reshape(1, 2))

    s2 = scat(p2p, src_p, dst_p, zeros_nw)

    out = pl.pallas_call(
        _tc_final_body,
        out_shape=jax.ShapeDtypeStruct((n, W2.shape[1]), jnp.float32),
    )(s2, p2p, dinv, W2, b2.reshape(1, W2.shape[1]))
    return out


# speedup vs baseline: 36.7490x; 1.1063x over previous
import functools

import jax
import jax.numpy as jnp
from jax import lax
from jax.experimental import pallas as pl
from jax.experimental.pallas import tpu as pltpu
from jax.experimental.pallas import tpu_sc as plsc

_NC = 2
_NS = 16
_CHUNK = 128
_W = 16


def _sc_degree(n_pad, n_chunks):
    mesh = plsc.VectorSubcoreMesh(core_axis_name="c", subcore_axis_name="s")

    @functools.partial(
        pl.kernel,
        out_type=jax.ShapeDtypeStruct((_NC, n_pad, _W), jnp.float32),
        mesh=mesh,
        compiler_params=pltpu.CompilerParams(use_tc_tiling_on_sc=False),
        scratch_types=[
            pltpu.VMEM((n_chunks, _CHUNK), jnp.int32),
            pltpu.VMEM((_CHUNK, _W), jnp.float32),
            pltpu.VMEM_SHARED((n_pad, _W), jnp.float32),
        ],
    )
    def deg_kernel(ones_hbm, dst_hbm, init_hbm, out_hbm, didx, rows, acc):
        c = lax.axis_index("c")
        s = lax.axis_index("s")

        @pl.when(s == 0)
        def _():
            pltpu.sync_copy(init_hbm, acc)

        w = c * _NS + s
        pltpu.sync_copy(dst_hbm.at[w], didx)
        pltpu.sync_copy(ones_hbm, rows)
        plsc.subcore_barrier()

        def step(i, carry):
            pltpu.sync_copy(rows, acc.at[didx.at[i]], add=True)
            return carry

        lax.fori_loop(0, n_chunks, step, 0)
        plsc.subcore_barrier()

        @pl.when(s == 0)
        def _():
            pltpu.sync_copy(acc, out_hbm.at[c])

    return deg_kernel


def _sc_scatter(n_pad, n_chunks):
    mesh = plsc.VectorSubcoreMesh(core_axis_name="c", subcore_axis_name="s")
    depth = 8
    assert n_chunks % depth == 0

    @functools.partial(
        pl.kernel,
        out_type=jax.ShapeDtypeStruct((_NC, n_pad, _W), jnp.float32),
        mesh=mesh,
        compiler_params=pltpu.CompilerParams(use_tc_tiling_on_sc=False),
        scratch_types=[
            pltpu.VMEM((n_chunks, _CHUNK), jnp.int32),
            pltpu.VMEM((n_chunks, _CHUNK), jnp.int32),
            [pltpu.VMEM((_CHUNK, _W), jnp.float32) for _ in range(depth)],
            [pltpu.SemaphoreType.DMA for _ in range(depth)],
            pltpu.VMEM_SHARED((n_pad, _W), jnp.float32),
        ],
    )
    def scat_kernel(table_hbm, src_hbm, dst_hbm, init_hbm, out_hbm,
                    sidx, didx, rows, sems, acc):
        c = lax.axis_index("c")
        s = lax.axis_index("s")

        @pl.when(s == 0)
        def _():
            pltpu.sync_copy(init_hbm, acc)

        w = c * _NS + s
        pltpu.sync_copy(src_hbm.at[w], sidx)
        pltpu.sync_copy(dst_hbm.at[w], didx)
        plsc.subcore_barrier()

        for k in range(depth):
            pltpu.async_copy(table_hbm.at[sidx.at[k]], rows[k], sems[k])

        def step(q, carry):
            base = q * depth
            for k in range(depth):
                i = base + k
                pltpu.make_async_copy(table_hbm.at[sidx.at[i]], rows[k],
                                      sems[k]).wait()
                pltpu.sync_copy(rows[k], acc.at[didx.at[i]], add=True)

                @pl.when(i + depth < n_chunks)
                def _():
                    pltpu.async_copy(table_hbm.at[sidx.at[i + depth]],
                                     rows[k], sems[k])
            return carry

        lax.fori_loop(0, n_chunks // depth, step, 0)
        plsc.subcore_barrier()

        @pl.when(s == 0)
        def _():
            pltpu.sync_copy(acc, out_hbm.at[c])

    return scat_kernel


def _tc_prep_body(degp_ref, x_ref, w1_ref, p1_ref, dinv_ref):
    n = x_ref.shape[0]
    deg = degp_ref[0, :n, 0:1] + degp_ref[1, :n, 0:1] + 1.0
    dinv = lax.rsqrt(deg)
    h1 = jnp.dot(x_ref[...], w1_ref[...], preferred_element_type=jnp.float32)
    p1_ref[...] = jnp.zeros(p1_ref.shape, jnp.float32)
    p1_ref[:n, 0:2] = h1 * dinv
    dinv_ref[...] = dinv


def _tc_mid_body(s1_ref, p1_ref, dinv_ref, b1_ref, p2_ref):
    n = dinv_ref.shape[0]
    stot = s1_ref[0, :n, 0:2] + s1_ref[1, :n, 0:2] + p1_ref[:n, 0:2]
    z = jnp.maximum(dinv_ref[...] * stot + b1_ref[...], 0.0)
    p2_ref[...] = jnp.zeros(p2_ref.shape, jnp.float32)
    p2_ref[:n, 0:2] = dinv_ref[...] * z


def _tc_final_body(s2_ref, p2_ref, dinv_ref, w2_ref, b2_ref, out_ref):
    n = dinv_ref.shape[0]
    agg = dinv_ref[...] * (
        s2_ref[0, :n, 0:2] + s2_ref[1, :n, 0:2] + p2_ref[:n, 0:2])
    y = jnp.dot(agg, w2_ref[...], preferred_element_type=jnp.float32)
    y = y + b2_ref[...]
    m = jnp.max(y, axis=-1, keepdims=True)
    e = jnp.exp(y - m)
    out_ref[...] = (y - m) - jnp.log(jnp.sum(e, axis=-1, keepdims=True))


def kernel(x, edge_index, W1, b1, W2, b2):
    n, d_in = x.shape
    e = edge_index.shape[1]
    n_pad = n + 8
    workers = _NC * _NS
    step = workers * _CHUNK * 4
    e_pad = ((e + step - 1) // step) * step
    n_chunks = e_pad // (workers * _CHUNK)

    src = edge_index[0]
    dst = edge_index[1]
    fill = (jnp.arange(e_pad - e, dtype=src.dtype) % 8) + n
    src_p = jnp.concatenate([src, fill]).reshape(workers, n_chunks, _CHUNK)
    dst_p = jnp.concatenate([dst, fill]).reshape(workers, n_chunks, _CHUNK)

    zeros_nw = jnp.zeros((n_pad, _W), jnp.float32)
    ones = jnp.ones((_CHUNK, _W), jnp.float32)

    deg_parts = _sc_degree(n_pad, n_chunks)(ones, dst_p, zeros_nw)

    p1p, dinv = pl.pallas_call(
        _tc_prep_body,
        out_shape=(jax.ShapeDtypeStruct((n_pad, _W), jnp.float32),
                   jax.ShapeDtypeStruct((n, 1), jnp.float32)),
    )(deg_parts, x, W1)

    scat = _sc_scatter(n_pad, n_chunks)
    s1 = scat(p1p, src_p, dst_p, zeros_nw)

    p2p = pl.pallas_call(
        _tc_mid_body,
        out_shape=jax.ShapeDtypeStruct((n_pad, _W), jnp.float32),
    )(s1, p1p, dinv, b1.reshape(1, 2))

    s2 = scat(p2p, src_p, dst_p, zeros_nw)

    out = pl.pallas_call(
        _tc_final_body,
        out_shape=jax.ShapeDtypeStruct((n, W2.shape[1]), jnp.float32),
    )(s2, p2p, dinv, W2, b2.reshape(1, W2.shape[1]))
    return out


# speedup vs baseline: 59.6931x; 1.6243x over previous
import functools

import jax
import jax.numpy as jnp
from jax import lax
from jax.experimental import pallas as pl
from jax.experimental.pallas import tpu as pltpu
from jax.experimental.pallas import tpu_sc as plsc

_NC = 2
_NS = 16
_CHUNK = 128
_W = 16


def _sc_degree(n_pad, n_chunks):
    mesh = plsc.VectorSubcoreMesh(core_axis_name="c", subcore_axis_name="s")

    @functools.partial(
        pl.kernel,
        out_type=jax.ShapeDtypeStruct((_NC, n_pad, _W), jnp.float32),
        mesh=mesh,
        compiler_params=pltpu.CompilerParams(use_tc_tiling_on_sc=False),
        scratch_types=[
            pltpu.VMEM((n_chunks, _CHUNK), jnp.int32),
            pltpu.VMEM((_CHUNK, _W), jnp.float32),
            pltpu.VMEM_SHARED((n_pad, _W), jnp.float32),
        ],
    )
    def deg_kernel(ones_hbm, dst_hbm, init_hbm, out_hbm, didx, rows, acc):
        c = lax.axis_index("c")
        s = lax.axis_index("s")

        @pl.when(s == 0)
        def _():
            pltpu.sync_copy(init_hbm, acc)

        w = c * _NS + s
        pltpu.sync_copy(dst_hbm.at[w], didx)
        pltpu.sync_copy(ones_hbm, rows)
        plsc.subcore_barrier()

        def step(i, carry):
            pltpu.sync_copy(rows, acc.at[didx.at[i]], add=True)
            return carry

        lax.fori_loop(0, n_chunks, step, 0)
        plsc.subcore_barrier()

        @pl.when(s == 0)
        def _():
            pltpu.sync_copy(acc, out_hbm.at[c])

    return deg_kernel


def _sc_scatter(n_pad, n_chunks):
    mesh = plsc.VectorSubcoreMesh(core_axis_name="c", subcore_axis_name="s")
    depth = 8
    assert n_chunks % depth == 0

    @functools.partial(
        pl.kernel,
        out_type=jax.ShapeDtypeStruct((_NC, n_pad, _W), jnp.float32),
        mesh=mesh,
        compiler_params=pltpu.CompilerParams(use_tc_tiling_on_sc=False),
        scratch_types=[
            pltpu.VMEM((n_chunks, _CHUNK), jnp.int32),
            pltpu.VMEM((n_chunks, _CHUNK), jnp.int32),
            [pltpu.VMEM((_CHUNK, _W), jnp.float32) for _ in range(depth)],
            [pltpu.SemaphoreType.DMA for _ in range(depth)],
            pltpu.VMEM_SHARED((n_pad, _W), jnp.float32),
            pltpu.VMEM_SHARED((n_pad, _W), jnp.float32),
        ],
    )
    def scat_kernel(table_hbm, src_hbm, dst_hbm, init_hbm, out_hbm,
                    sidx, didx, rows, sems, acc, table_s):
        c = lax.axis_index("c")
        s = lax.axis_index("s")

        @pl.when(s == 0)
        def _():
            pltpu.sync_copy(init_hbm, acc)

        @pl.when(s == 1)
        def _():
            pltpu.sync_copy(table_hbm, table_s)

        w = c * _NS + s
        pltpu.sync_copy(src_hbm.at[w], sidx)
        pltpu.sync_copy(dst_hbm.at[w], didx)
        plsc.subcore_barrier()

        for k in range(depth):
            pltpu.async_copy(table_s.at[sidx.at[k]], rows[k], sems[k])

        def step(q, carry):
            base = q * depth
            for k in range(depth):
                i = base + k
                pltpu.make_async_copy(table_s.at[sidx.at[i]], rows[k],
                                      sems[k]).wait()
                pltpu.sync_copy(rows[k], acc.at[didx.at[i]], add=True)

                @pl.when(i + depth < n_chunks)
                def _():
                    pltpu.async_copy(table_s.at[sidx.at[i + depth]],
                                     rows[k], sems[k])
            return carry

        lax.fori_loop(0, n_chunks // depth, step, 0)
        plsc.subcore_barrier()

        @pl.when(s == 0)
        def _():
            pltpu.sync_copy(acc, out_hbm.at[c])

    return scat_kernel


def _tc_prep_body(degp_ref, x_ref, w1_ref, p1_ref, dinv_ref):
    n = x_ref.shape[0]
    deg = degp_ref[0, :n, 0:1] + degp_ref[1, :n, 0:1] + 1.0
    dinv = lax.rsqrt(deg)
    h1 = jnp.dot(x_ref[...], w1_ref[...], preferred_element_type=jnp.float32)
    p1_ref[...] = jnp.zeros(p1_ref.shape, jnp.float32)
    p1_ref[:n, 0:2] = h1 * dinv
    dinv_ref[...] = dinv


def _tc_mid_body(s1_ref, p1_ref, dinv_ref, b1_ref, p2_ref):
    n = dinv_ref.shape[0]
    stot = s1_ref[0, :n, 0:2] + s1_ref[1, :n, 0:2] + p1_ref[:n, 0:2]
    z = jnp.maximum(dinv_ref[...] * stot + b1_ref[...], 0.0)
    p2_ref[...] = jnp.zeros(p2_ref.shape, jnp.float32)
    p2_ref[:n, 0:2] = dinv_ref[...] * z


def _tc_final_body(s2_ref, p2_ref, dinv_ref, w2_ref, b2_ref, out_ref):
    n = dinv_ref.shape[0]
    agg = dinv_ref[...] * (
        s2_ref[0, :n, 0:2] + s2_ref[1, :n, 0:2] + p2_ref[:n, 0:2])
    y = jnp.dot(agg, w2_ref[...], preferred_element_type=jnp.float32)
    y = y + b2_ref[...]
    m = jnp.max(y, axis=-1, keepdims=True)
    e = jnp.exp(y - m)
    out_ref[...] = (y - m) - jnp.log(jnp.sum(e, axis=-1, keepdims=True))


def kernel(x, edge_index, W1, b1, W2, b2):
    n, d_in = x.shape
    e = edge_index.shape[1]
    n_pad = n + 8
    workers = _NC * _NS
    step = workers * _CHUNK * 4
    e_pad = ((e + step - 1) // step) * step
    n_chunks = e_pad // (workers * _CHUNK)

    src = edge_index[0]
    dst = edge_index[1]
    fill = (jnp.arange(e_pad - e, dtype=src.dtype) % 8) + n
    src_p = jnp.concatenate([src, fill]).reshape(workers, n_chunks, _CHUNK)
    dst_p = jnp.concatenate([dst, fill]).reshape(workers, n_chunks, _CHUNK)

    zeros_nw = jnp.zeros((n_pad, _W), jnp.float32)
    ones = jnp.ones((_CHUNK, _W), jnp.float32)

    deg_parts = _sc_degree(n_pad, n_chunks)(ones, dst_p, zeros_nw)

    p1p, dinv = pl.pallas_call(
        _tc_prep_body,
        out_shape=(jax.ShapeDtypeStruct((n_pad, _W), jnp.float32),
                   jax.ShapeDtypeStruct((n, 1), jnp.float32)),
    )(deg_parts, x, W1)

    scat = _sc_scatter(n_pad, n_chunks)
    s1 = scat(p1p, src_p, dst_p, zeros_nw)

    p2p = pl.pallas_call(
        _tc_mid_body,
        out_shape=jax.ShapeDtypeStruct((n_pad, _W), jnp.float32),
    )(s1, p1p, dinv, b1.reshape(1, 2))

    s2 = scat(p2p, src_p, dst_p, zeros_nw)

    out = pl.pallas_call(
        _tc_final_body,
        out_shape=jax.ShapeDtypeStruct((n, W2.shape[1]), jnp.float32),
    )(s2, p2p, dinv, W2, b2.reshape(1, W2.shape[1]))
    return out
